# baseline (speedup 1.0000x reference)
import functools

import jax
import jax.numpy as jnp
from jax import lax
from jax.experimental import pallas as pl
from jax.experimental.pallas import tpu as pltpu
from jax.experimental.pallas import tpu_sc as plsc

_N, _D, _E = 10000, 128, 320000
_NC, _NS = 2, 16
_NW = _NC * _NS
_EPW = _E // _NW
_CH = 128
_NCH = _EPW // _CH
_TL = _EPW - _NCH * _CH
_NPAIR = _NCH // 2
_NP = 10240
_RPT = _NP // _NS
_BLK = 1024
_GRID = 10

_mesh = plsc.VectorSubcoreMesh(
    core_axis_name="c", subcore_axis_name="s", num_cores=_NC, num_subcores=_NS
)


_HP = _NP + 16


@functools.partial(
    pl.kernel,
    out_type=(
        jax.ShapeDtypeStruct((_NC * _NP, _D), jnp.float32),
        jax.ShapeDtypeStruct((_NC * _NP,), jnp.float32),
        jax.ShapeDtypeStruct((_NW * _NP,), jnp.float32),
    ),
    mesh=_mesh,
    scratch_types=[
        pltpu.VMEM((2 * _CH,), jnp.int32),
        pltpu.VMEM((2 * _CH,), jnp.int32),
        pltpu.VMEM((2 * _CH,), jnp.int32),
        pltpu.VMEM((2 * _CH,), jnp.int32),
        pltpu.VMEM((_CH, _D), jnp.float32),
        pltpu.VMEM((_CH, _D), jnp.float32),
        pltpu.VMEM((_TL,), jnp.int32),
        pltpu.VMEM((_TL,), jnp.int32),
        pltpu.VMEM((_TL, _D), jnp.float32),
        pltpu.VMEM((_HP,), jnp.float32),
        pltpu.VMEM((_RPT,), jnp.float32),
        pltpu.VMEM((_RPT,), jnp.float32),
        pltpu.VMEM_SHARED((_NP, _D), jnp.float32),
        pltpu.SemaphoreType.DMA,
        pltpu.SemaphoreType.DMA,
        pltpu.SemaphoreType.DMA,
        pltpu.SemaphoreType.DMA,
        pltpu.SemaphoreType.DMA,
        pltpu.SemaphoreType.DMA,
        pltpu.SemaphoreType.DMA,
        pltpu.SemaphoreType.DMA,
    ],
)
def _sc_segment_mean(y_hbm, src_hbm, dst_hbm, zrow_hbm,
                     acc_out, cnt_out, stage_out,
                     sidx_a, sidx_b, didx_a, didx_b, rows_a, rows_b,
                     sidx_t, didx_t, rows_t, hist, racc, rtmp, acc_sh,
                     sem_sa, sem_sb, sem_da, sem_db, sem_ga, sem_gb,
                     sem_xa, sem_xb):
    c = lax.axis_index("c")
    s = lax.axis_index("s")
    wid = s * _NC + c
    r0 = s * _RPT
    nz = _RPT // _CH
    ebase = wid * _EPW
    one16 = jnp.where(lax.iota(jnp.int32, 16) == 0,
                      jnp.float32(1.0), jnp.float32(0.0))
    z16 = jnp.zeros((16,), jnp.float32)

    def idx_issue(p, sbuf, dbuf, sem_s, sem_d):
        b = ebase + p * (2 * _CH)
        pltpu.async_copy(src_hbm.at[pl.ds(b, 2 * _CH)], sbuf, sem_s)
        pltpu.async_copy(dst_hbm.at[pl.ds(b, 2 * _CH)], dbuf, sem_d)

    def idx_wait(sbuf, dbuf, sem_s, sem_d):
        pltpu.make_async_copy(src_hbm.at[pl.ds(0, 2 * _CH)], sbuf, sem_s).wait()
        pltpu.make_async_copy(dst_hbm.at[pl.ds(0, 2 * _CH)], dbuf, sem_d).wait()

    def gather_issue(sbuf, rbuf, sem_g):
        pltpu.async_copy(y_hbm.at[sbuf], rbuf, sem_g)

    def gather_wait(sbuf, rbuf, sem_g):
        pltpu.make_async_copy(y_hbm.at[sbuf], rbuf, sem_g).wait()

    def count(dbuf):
        for g in range(_CH // 16):
            dv = dbuf[pl.ds(g * 16, 16)]
            for l in range(16):
                d = dv[l]
                hist[pl.ds(d, 16)] = hist[pl.ds(d, 16)] + one16

    pltpu.sync_copy(zrow_hbm, rows_a)
    for k in range(nz):
        pltpu.sync_copy(rows_a, acc_sh.at[pl.ds(r0 + k * _CH, _CH)])

    def hz(i, carry):
        hist[pl.ds(i * 16, 16)] = z16
        return carry

    lax.fori_loop(0, _HP // 16, hz, 0)
    plsc.subcore_barrier()

    idx_issue(0, sidx_a, didx_a, sem_sa, sem_da)
    idx_wait(sidx_a, didx_a, sem_sa, sem_da)
    gather_issue(sidx_a.at[pl.ds(0, _CH)], rows_a, sem_ga)

    def halfpair(sidx_p, didx_p, h, rbuf, sem_g, sem_x, nsidx, nrbuf, nsem_g,
                 prefetch):
        sl = pl.ds(h * _CH, _CH)
        gather_wait(sidx_p.at[sl], rbuf, sem_g)
        if prefetch is not None:
            prefetch()
        gather_issue(nsidx, nrbuf, nsem_g)
        pltpu.async_copy(rbuf, acc_sh.at[didx_p.at[sl]], sem_x, add=True)
        for g in range(_CH // 16):
            dv = didx_p[pl.ds(h * _CH + g * 16, 16)]
            for l in range(16):
                d = dv[l]
                hist[pl.ds(d, 16)] = hist[pl.ds(d, 16)] + one16
        pltpu.make_async_copy(rbuf, acc_sh.at[didx_p.at[sl]], sem_x).wait()

    def pair(i, carry):
        p = 2 * i

        def pf_b():
            idx_issue(jnp.minimum(p + 1, _NPAIR - 1), sidx_b, didx_b,
                      sem_sb, sem_db)

        def wt_b():
            idx_wait(sidx_b, didx_b, sem_sb, sem_db)

        def pf_a():
            idx_issue(jnp.minimum(p + 2, _NPAIR - 1), sidx_a, didx_a,
                      sem_sa, sem_da)

        def wt_a():
            idx_wait(sidx_a, didx_a, sem_sa, sem_da)

        halfpair(sidx_a, didx_a, 0, rows_a, sem_ga, sem_xa,
                 sidx_a.at[pl.ds(_CH, _CH)], rows_b, sem_gb, pf_b)
        halfpair(sidx_a, didx_a, 1, rows_b, sem_gb, sem_xb,
                 sidx_b.at[pl.ds(0, _CH)], rows_a, sem_ga, wt_b)
        halfpair(sidx_b, didx_b, 0, rows_a, sem_ga, sem_xa,
                 sidx_b.at[pl.ds(_CH, _CH)], rows_b, sem_gb, pf_a)
        halfpair(sidx_b, didx_b, 1, rows_b, sem_gb, sem_xb,
                 sidx_a.at[pl.ds(0, _CH)], rows_a, sem_ga, wt_a)
        return carry

    lax.fori_loop(0, _NPAIR // 2, pair, 0)
    halfpair(sidx_a, didx_a, 0, rows_a, sem_ga, sem_xa,
             sidx_a.at[pl.ds(_CH, _CH)], rows_b, sem_gb, None)
    halfpair(sidx_a, didx_a, 1, rows_b, sem_gb, sem_xb,
             sidx_a.at[pl.ds(0, _CH)], rows_a, sem_ga, None)
    gather_wait(sidx_a.at[pl.ds(0, _CH)], rows_a, sem_ga)
    bt = ebase + _NCH * _CH
    pltpu.sync_copy(src_hbm.at[pl.ds(bt, _TL)], sidx_t)
    pltpu.sync_copy(dst_hbm.at[pl.ds(bt, _TL)], didx_t)
    pltpu.async_copy(y_hbm.at[sidx_t], rows_t, sem_ga).wait()
    pltpu.sync_copy(rows_t, acc_sh.at[didx_t], add=True)
    for l in range(_TL):
        dv = didx_t[pl.ds(0, 16)]
        d = dv[l]
        hist[pl.ds(d, 16)] = hist[pl.ds(d, 16)] + one16

    pltpu.sync_copy(hist.at[pl.ds(0, _NP)], stage_out.at[pl.ds(wid * _NP, _NP)])
    plsc.subcore_barrier()

    for k in range(nz):
        pltpu.sync_copy(acc_sh.at[pl.ds(r0 + k * _CH, _CH)], rows_a)
        pltpu.sync_copy(rows_a, acc_out.at[pl.ds(c * _NP + r0 + k * _CH, _CH)])

    def rz(i, carry):
        racc[pl.ds(i * 16, 16)] = z16
        return carry

    lax.fori_loop(0, _RPT // 16, rz, 0)
    for t in range(_NS):
        twid = t * _NC + c
        pltpu.sync_copy(stage_out.at[pl.ds(twid * _NP + r0, _RPT)], rtmp)

        def radd(i, carry):
            sl = pl.ds(i * 16, 16)
            racc[sl] = racc[sl] + rtmp[sl]
            return carry

        lax.fori_loop(0, _RPT // 16, radd, 0)
    pltpu.sync_copy(racc, cnt_out.at[pl.ds(c * _NP + r0, _RPT)])


def _dotT(x, w):
    return lax.dot_general(x, w, dimension_numbers=(((1,), (1,)), ((), ())),
                           preferred_element_type=jnp.float32)


def _linear_body(xa, xf, xi, xp, wa, wf, wi, wp, ba, bf, bi, bp,
                 w1, w2, w3, w4, w5, w6, w7,
                 oa, of, oi, op, y1, y2, y3, y4, y5, y6, y7):
    a, f, i, p = xa[...], xf[...], xi[...], xp[...]
    oa[...] = _dotT(a, wa[...]) + ba[...]
    of[...] = _dotT(f, wf[...]) + bf[...]
    oi[...] = _dotT(i, wi[...]) + bi[...]
    op[...] = _dotT(p, wp[...]) + bp[...]
    y1[...] = _dotT(a, w1[...])
    y2[...] = _dotT(i, w2[...])
    y3[...] = _dotT(a, w3[...])
    y4[...] = _dotT(p, w4[...])
    y5[...] = _dotT(p, w5[...])
    y6[...] = _dotT(p, w6[...])
    y7[...] = _dotT(f, w7[...])


_xspec = pl.BlockSpec((_BLK, _D), lambda i: (i, 0))
_wspec = pl.BlockSpec((_D, _D), lambda i: (0, 0))
_bspec = pl.BlockSpec((1, _D), lambda i: (0, 0))
_accspec = pl.BlockSpec((_NC, _BLK, _D), lambda i: (0, i, 0))
_oshape = jax.ShapeDtypeStruct((_N, _D), jnp.float32)

_linear_call = pl.pallas_call(
    _linear_body,
    grid=(_GRID,),
    in_specs=[_xspec] * 4 + [_wspec] * 4 + [_bspec] * 4 + [_wspec] * 7,
    out_specs=[_xspec] * 11,
    out_shape=[_oshape] * 11,
)


def _agg(acc_ref, cnt_ref):
    acc = acc_ref[...]
    total = acc[0] + acc[1]
    n = cnt_ref[0, :] + cnt_ref[1, :]
    return total / jnp.maximum(n, 1.0)[:, None]


def _epilogue_body(ra, rf, ri, rp, a1, a2, a3, a4, a5, a6, a7,
                   c1, c2, c3, c4, c5, c6, c7, oa, of, oi, op):
    oa[...] = ra[...] + _agg(a2, c2) + _agg(a4, c4)
    of[...] = rf[...] + _agg(a6, c6)
    oi[...] = ri[...] + _agg(a1, c1)
    op[...] = rp[...] + _agg(a3, c3) + _agg(a5, c5) + _agg(a7, c7)


_cntspec = pl.BlockSpec((_NC, _BLK), lambda i: (0, i))

_epilogue_call = pl.pallas_call(
    _epilogue_body,
    grid=(_GRID,),
    in_specs=[_xspec] * 4 + [_accspec] * 7 + [_cntspec] * 7,
    out_specs=[_xspec] * 4,
    out_shape=[_oshape] * 4,
)


def kernel(x_author, W_root_author, b_root_author,
           x_field_of_study, W_root_field_of_study, b_root_field_of_study,
           x_institution, W_root_institution, b_root_institution,
           x_paper, W_root_paper, b_root_paper,
           W_rel_author_affiliated_with_institution, ei_author_affiliated_with_institution,
           W_rel_institution_to_author, ei_institution_to_author,
           W_rel_author_writes_paper, ei_author_writes_paper,
           W_rel_paper_to_author, ei_paper_to_author,
           W_rel_paper_cites_paper, ei_paper_cites_paper,
           W_rel_paper_has_topic_field_of_study, ei_paper_has_topic_field_of_study,
           W_rel_field_of_study_to_paper, ei_field_of_study_to_paper):
    outs = _linear_call(
        x_author, x_field_of_study, x_institution, x_paper,
        W_root_author, W_root_field_of_study, W_root_institution, W_root_paper,
        b_root_author.reshape(1, _D), b_root_field_of_study.reshape(1, _D),
        b_root_institution.reshape(1, _D), b_root_paper.reshape(1, _D),
        W_rel_author_affiliated_with_institution, W_rel_institution_to_author,
        W_rel_author_writes_paper, W_rel_paper_to_author, W_rel_paper_cites_paper,
        W_rel_paper_has_topic_field_of_study, W_rel_field_of_study_to_paper,
    )
    roots = outs[:4]
    ys = outs[4:]
    eis = (ei_author_affiliated_with_institution, ei_institution_to_author,
           ei_author_writes_paper, ei_paper_to_author, ei_paper_cites_paper,
           ei_paper_has_topic_field_of_study, ei_field_of_study_to_paper)

    zrow = jnp.zeros((_CH, _D), jnp.float32)

    accs, cnts = [], []
    for y, ei in zip(ys, eis):
        acc, cnt, _ = _sc_segment_mean(y, ei[1], ei[0], zrow)
        accs.append(acc.reshape(_NC, _NP, _D))
        cnts.append(cnt.reshape(_NC, _NP))

    return tuple(_epilogue_call(*roots, *accs, *cnts))


# speedup vs baseline: 1.0504x; 1.0504x over previous
import functools

import jax
import jax.numpy as jnp
from jax import lax
from jax.experimental import pallas as pl
from jax.experimental.pallas import tpu as pltpu
from jax.experimental.pallas import tpu_sc as plsc

_N, _D, _E = 10000, 128, 320000
_NC, _NS = 2, 16
_NW = _NC * _NS
_EPW = _E // _NW
_CH = 128
_NCH = _EPW // _CH
_TL = _EPW - _NCH * _CH
_NPAIR = _NCH // 2
_NP = 10240
_RPT = _NP // _NS
_BLK = 1024
_GRID = 10

_mesh = plsc.VectorSubcoreMesh(
    core_axis_name="c", subcore_axis_name="s", num_cores=_NC, num_subcores=_NS
)


_HP = _NP + 16


@functools.partial(
    pl.kernel,
    out_type=(
        jax.ShapeDtypeStruct((_NC * _NP, _D), jnp.float32),
        jax.ShapeDtypeStruct((_NC * _NP,), jnp.float32),
        jax.ShapeDtypeStruct((_NW * _NP,), jnp.float32),
    ),
    mesh=_mesh,
    scratch_types=[
        pltpu.VMEM((_CH,), jnp.int32),
        pltpu.VMEM((_CH,), jnp.int32),
        pltpu.VMEM((_CH,), jnp.int32),
        pltpu.VMEM((_CH,), jnp.int32),
        pltpu.VMEM((_CH, _D), jnp.float32),
        pltpu.VMEM((_CH, _D), jnp.float32),
        pltpu.VMEM((_TL,), jnp.int32),
        pltpu.VMEM((_TL,), jnp.int32),
        pltpu.VMEM((_TL, _D), jnp.float32),
        pltpu.VMEM((_HP,), jnp.float32),
        pltpu.VMEM((_RPT,), jnp.float32),
        pltpu.VMEM((_RPT,), jnp.float32),
        pltpu.VMEM((_RPT,), jnp.float32),
        pltpu.VMEM_SHARED((_NP, _D), jnp.float32),
        pltpu.SemaphoreType.DMA,
        pltpu.SemaphoreType.DMA,
        pltpu.SemaphoreType.DMA,
        pltpu.SemaphoreType.DMA,
        pltpu.SemaphoreType.DMA,
        pltpu.SemaphoreType.DMA,
        pltpu.SemaphoreType.DMA,
        pltpu.SemaphoreType.DMA,
    ],
)
def _sc_segment_mean(y_hbm, src_hbm, dst_hbm, zrow_hbm, zflat_hbm,
                     acc_out, cnt_out, stage_out,
                     sidx_a, sidx_b, didx_a, didx_b, rows_a, rows_b,
                     sidx_t, didx_t, rows_t, hist, racc, rtmp, rtmp2, acc_sh,
                     sem_sa, sem_sb, sem_da, sem_db, sem_ga, sem_gb,
                     sem_xa, sem_xb):
    c = lax.axis_index("c")
    s = lax.axis_index("s")
    wid = s * _NC + c
    r0 = s * _RPT
    nz = _RPT // _CH
    ebase = wid * _EPW
    one16 = jnp.where(lax.iota(jnp.int32, 16) == 0,
                      jnp.float32(1.0), jnp.float32(0.0))
    z16 = jnp.zeros((16,), jnp.float32)

    def idx_issue(k, sbuf, dbuf, sem_s, sem_d):
        b = ebase + k * _CH
        pltpu.async_copy(src_hbm.at[pl.ds(b, _CH)], sbuf, sem_s)
        pltpu.async_copy(dst_hbm.at[pl.ds(b, _CH)], dbuf, sem_d)

    def idx_wait(sbuf, dbuf, sem_s, sem_d):
        pltpu.make_async_copy(src_hbm.at[pl.ds(0, _CH)], sbuf, sem_s).wait()
        pltpu.make_async_copy(dst_hbm.at[pl.ds(0, _CH)], dbuf, sem_d).wait()

    def gather_issue(sbuf, rbuf, sem_g):
        pltpu.async_copy(y_hbm.at[sbuf], rbuf, sem_g)

    def gather_wait(sbuf, rbuf, sem_g):
        pltpu.make_async_copy(y_hbm.at[sbuf], rbuf, sem_g).wait()

    def count(dbuf):
        for g in range(_CH // 16):
            dv = dbuf[pl.ds(g * 16, 16)]
            for l in range(16):
                d = dv[l]
                hist[pl.ds(d, 16)] = hist[pl.ds(d, 16)] + one16

    pltpu.sync_copy(zrow_hbm, rows_a)
    for k in range(nz):
        pltpu.sync_copy(rows_a, acc_sh.at[pl.ds(r0 + k * _CH, _CH)])

    pltpu.sync_copy(zflat_hbm, hist)
    plsc.subcore_barrier()

    idx_issue(0, sidx_a, didx_a, sem_sa, sem_da)
    idx_wait(sidx_a, didx_a, sem_sa, sem_da)
    gather_issue(sidx_a, rows_a, sem_ga)

    def pair(i, carry):
        a = 2 * i
        idx_issue(a + 1, sidx_b, didx_b, sem_sb, sem_db)
        gather_wait(sidx_a, rows_a, sem_ga)
        idx_wait(sidx_b, didx_b, sem_sb, sem_db)
        gather_issue(sidx_b, rows_b, sem_gb)
        pltpu.async_copy(rows_a, acc_sh.at[didx_a], sem_xa, add=True)
        count(didx_a)
        pltpu.make_async_copy(rows_a, acc_sh.at[didx_a], sem_xa).wait()
        idx_issue(jnp.minimum(a + 2, _NCH - 1), sidx_a, didx_a, sem_sa, sem_da)
        gather_wait(sidx_b, rows_b, sem_gb)
        idx_wait(sidx_a, didx_a, sem_sa, sem_da)
        gather_issue(sidx_a, rows_a, sem_ga)
        pltpu.async_copy(rows_b, acc_sh.at[didx_b], sem_xb, add=True)
        count(didx_b)
        pltpu.make_async_copy(rows_b, acc_sh.at[didx_b], sem_xb).wait()
        return carry

    lax.fori_loop(0, _NPAIR, pair, 0)
    gather_wait(sidx_a, rows_a, sem_ga)
    bt = ebase + _NCH * _CH
    pltpu.sync_copy(src_hbm.at[pl.ds(bt, _TL)], sidx_t)
    pltpu.sync_copy(dst_hbm.at[pl.ds(bt, _TL)], didx_t)
    pltpu.async_copy(y_hbm.at[sidx_t], rows_t, sem_ga).wait()
    pltpu.sync_copy(rows_t, acc_sh.at[didx_t], add=True)
    for l in range(_TL):
        dv = didx_t[pl.ds(0, 16)]
        d = dv[l]
        hist[pl.ds(d, 16)] = hist[pl.ds(d, 16)] + one16

    pltpu.sync_copy(hist.at[pl.ds(0, _NP)], stage_out.at[pl.ds(wid * _NP, _NP)])
    plsc.subcore_barrier()

    bufs = [rows_a, rows_b]
    sems = [sem_xa, sem_xb]
    for k in range(nz):
        bk = bufs[k % 2]
        if k >= 2:
            pltpu.make_async_copy(
                bk, acc_out.at[pl.ds(c * _NP + r0 + (k - 2) * _CH, _CH)],
                sems[k % 2]).wait()
        pltpu.sync_copy(acc_sh.at[pl.ds(r0 + k * _CH, _CH)], bk)
        pltpu.async_copy(bk, acc_out.at[pl.ds(c * _NP + r0 + k * _CH, _CH)],
                         sems[k % 2])
    for k in range(nz - 2, nz):
        pltpu.make_async_copy(
            bufs[k % 2], acc_out.at[pl.ds(c * _NP + r0 + k * _CH, _CH)],
            sems[k % 2]).wait()

    def rz(i, carry):
        racc[pl.ds(i * 16, 16)] = z16
        return carry

    lax.fori_loop(0, _RPT // 16, rz, 0)
    rbufs = [rtmp, rtmp2]
    rsems = [sem_ga, sem_gb]
    pltpu.async_copy(stage_out.at[pl.ds((0 * _NC + c) * _NP + r0, _RPT)],
                     rbufs[0], rsems[0])
    for t in range(_NS):
        rb = rbufs[t % 2]
        pltpu.make_async_copy(stage_out.at[pl.ds(0, _RPT)], rb,
                              rsems[t % 2]).wait()
        if t + 1 < _NS:
            twid = (t + 1) * _NC + c
            pltpu.async_copy(stage_out.at[pl.ds(twid * _NP + r0, _RPT)],
                             rbufs[(t + 1) % 2], rsems[(t + 1) % 2])

        def radd(i, carry, rb=rb):
            sl = pl.ds(i * 16, 16)
            racc[sl] = racc[sl] + rb[sl]
            return carry

        lax.fori_loop(0, _RPT // 16, radd, 0)
    pltpu.sync_copy(racc, cnt_out.at[pl.ds(c * _NP + r0, _RPT)])


def _dotT(x, w):
    return lax.dot_general(x, w, dimension_numbers=(((1,), (1,)), ((), ())),
                           preferred_element_type=jnp.float32)


def _linear_body(xa, xf, xi, xp, wa, wf, wi, wp, ba, bf, bi, bp,
                 w1, w2, w3, w4, w5, w6, w7,
                 oa, of, oi, op, y1, y2, y3, y4, y5, y6, y7):
    a, f, i, p = xa[...], xf[...], xi[...], xp[...]
    oa[...] = _dotT(a, wa[...]) + ba[...]
    of[...] = _dotT(f, wf[...]) + bf[...]
    oi[...] = _dotT(i, wi[...]) + bi[...]
    op[...] = _dotT(p, wp[...]) + bp[...]
    y1[...] = _dotT(a, w1[...])
    y2[...] = _dotT(i, w2[...])
    y3[...] = _dotT(a, w3[...])
    y4[...] = _dotT(p, w4[...])
    y5[...] = _dotT(p, w5[...])
    y6[...] = _dotT(p, w6[...])
    y7[...] = _dotT(f, w7[...])


_xspec = pl.BlockSpec((_BLK, _D), lambda i: (i, 0))
_wspec = pl.BlockSpec((_D, _D), lambda i: (0, 0))
_bspec = pl.BlockSpec((1, _D), lambda i: (0, 0))
_accspec = pl.BlockSpec((_NC, _BLK, _D), lambda i: (0, i, 0))
_oshape = jax.ShapeDtypeStruct((_N, _D), jnp.float32)

_linear_call = pl.pallas_call(
    _linear_body,
    grid=(_GRID,),
    in_specs=[_xspec] * 4 + [_wspec] * 4 + [_bspec] * 4 + [_wspec] * 7,
    out_specs=[_xspec] * 11,
    out_shape=[_oshape] * 11,
)


def _agg(acc_ref, cnt_ref):
    acc = acc_ref[...]
    total = acc[0] + acc[1]
    n = cnt_ref[0, :] + cnt_ref[1, :]
    return total / jnp.maximum(n, 1.0)[:, None]


def _epilogue_body(ra, rf, ri, rp, a1, a2, a3, a4, a5, a6, a7,
                   c1, c2, c3, c4, c5, c6, c7, oa, of, oi, op):
    oa[...] = ra[...] + _agg(a2, c2) + _agg(a4, c4)
    of[...] = rf[...] + _agg(a6, c6)
    oi[...] = ri[...] + _agg(a1, c1)
    op[...] = rp[...] + _agg(a3, c3) + _agg(a5, c5) + _agg(a7, c7)


_cntspec = pl.BlockSpec((_NC, _BLK), lambda i: (0, i))

_epilogue_call = pl.pallas_call(
    _epilogue_body,
    grid=(_GRID,),
    in_specs=[_xspec] * 4 + [_accspec] * 7 + [_cntspec] * 7,
    out_specs=[_xspec] * 4,
    out_shape=[_oshape] * 4,
)


def kernel(x_author, W_root_author, b_root_author,
           x_field_of_study, W_root_field_of_study, b_root_field_of_study,
           x_institution, W_root_institution, b_root_institution,
           x_paper, W_root_paper, b_root_paper,
           W_rel_author_affiliated_with_institution, ei_author_affiliated_with_institution,
           W_rel_institution_to_author, ei_institution_to_author,
           W_rel_author_writes_paper, ei_author_writes_paper,
           W_rel_paper_to_author, ei_paper_to_author,
           W_rel_paper_cites_paper, ei_paper_cites_paper,
           W_rel_paper_has_topic_field_of_study, ei_paper_has_topic_field_of_study,
           W_rel_field_of_study_to_paper, ei_field_of_study_to_paper):
    outs = _linear_call(
        x_author, x_field_of_study, x_institution, x_paper,
        W_root_author, W_root_field_of_study, W_root_institution, W_root_paper,
        b_root_author.reshape(1, _D), b_root_field_of_study.reshape(1, _D),
        b_root_institution.reshape(1, _D), b_root_paper.reshape(1, _D),
        W_rel_author_affiliated_with_institution, W_rel_institution_to_author,
        W_rel_author_writes_paper, W_rel_paper_to_author, W_rel_paper_cites_paper,
        W_rel_paper_has_topic_field_of_study, W_rel_field_of_study_to_paper,
    )
    roots = outs[:4]
    ys = outs[4:]
    eis = (ei_author_affiliated_with_institution, ei_institution_to_author,
           ei_author_writes_paper, ei_paper_to_author, ei_paper_cites_paper,
           ei_paper_has_topic_field_of_study, ei_field_of_study_to_paper)

    zrow = jnp.zeros((_CH, _D), jnp.float32)
    zflat = jnp.zeros((_HP,), jnp.float32)

    accs, cnts = [], []
    for y, ei in zip(ys, eis):
        acc, cnt, _ = _sc_segment_mean(y, ei[1], ei[0], zrow, zflat)
        accs.append(acc.reshape(_NC, _NP, _D))
        cnts.append(cnt.reshape(_NC, _NP))

    return tuple(_epilogue_call(*roots, *accs, *cnts))


# speedup vs baseline: 1.0522x; 1.0017x over previous
import functools

import jax
import jax.numpy as jnp
from jax import lax
from jax.experimental import pallas as pl
from jax.experimental.pallas import tpu as pltpu
from jax.experimental.pallas import tpu_sc as plsc

_N, _D, _E = 10000, 128, 320000
_NC, _NS = 2, 16
_NW = _NC * _NS
_EPW = _E // _NW
_CH = 128
_NCH = _EPW // _CH
_TL = _EPW - _NCH * _CH
_NPAIR = _NCH // 2
_NP = 10240
_RPT = _NP // _NS
_BLK = 1024
_GRID = 10

_mesh = plsc.VectorSubcoreMesh(
    core_axis_name="c", subcore_axis_name="s", num_cores=_NC, num_subcores=_NS
)


_HP = _NP + 16


@functools.partial(
    pl.kernel,
    out_type=(
        jax.ShapeDtypeStruct((_NC * _NP, _D), jnp.float32),
        jax.ShapeDtypeStruct((_NC * _NP,), jnp.float32),
        jax.ShapeDtypeStruct((_NW * _NP,), jnp.float32),
    ),
    mesh=_mesh,
    scratch_types=[
        pltpu.VMEM((_CH,), jnp.int32),
        pltpu.VMEM((_CH,), jnp.int32),
        pltpu.VMEM((_CH,), jnp.int32),
        pltpu.VMEM((_CH,), jnp.int32),
        pltpu.VMEM((_CH, _D), jnp.float32),
        pltpu.VMEM((_CH, _D), jnp.float32),
        pltpu.VMEM((_TL,), jnp.int32),
        pltpu.VMEM((_TL,), jnp.int32),
        pltpu.VMEM((_TL, _D), jnp.float32),
        pltpu.VMEM((_HP,), jnp.float32),
        pltpu.VMEM((_RPT,), jnp.float32),
        pltpu.VMEM((_RPT,), jnp.float32),
        pltpu.VMEM((_RPT,), jnp.float32),
        pltpu.VMEM_SHARED((_NP, _D), jnp.float32),
        pltpu.SemaphoreType.DMA,
        pltpu.SemaphoreType.DMA,
        pltpu.SemaphoreType.DMA,
        pltpu.SemaphoreType.DMA,
        pltpu.SemaphoreType.DMA,
        pltpu.SemaphoreType.DMA,
        pltpu.SemaphoreType.DMA,
        pltpu.SemaphoreType.DMA,
    ],
)
def _sc_segment_mean(y_hbm, src_hbm, dst_hbm, zrow_hbm, zflat_hbm,
                     acc_out, cnt_out, stage_out,
                     sidx_a, sidx_b, didx_a, didx_b, rows_a, rows_b,
                     sidx_t, didx_t, rows_t, hist, racc, rtmp, rtmp2, acc_sh,
                     sem_sa, sem_sb, sem_da, sem_db, sem_ga, sem_gb,
                     sem_xa, sem_xb):
    c = lax.axis_index("c")
    s = lax.axis_index("s")
    wid = s * _NC + c
    r0 = s * _RPT
    nz = _RPT // _CH
    ebase = wid * _EPW
    one16 = jnp.where(lax.iota(jnp.int32, 16) == 0,
                      jnp.float32(1.0), jnp.float32(0.0))
    z16 = jnp.zeros((16,), jnp.float32)

    def idx_issue(k, sbuf, dbuf, sem_s, sem_d):
        b = ebase + k * _CH
        pltpu.async_copy(src_hbm.at[pl.ds(b, _CH)], sbuf, sem_s)
        pltpu.async_copy(dst_hbm.at[pl.ds(b, _CH)], dbuf, sem_d)

    def idx_wait(sbuf, dbuf, sem_s, sem_d):
        pltpu.make_async_copy(src_hbm.at[pl.ds(0, _CH)], sbuf, sem_s).wait()
        pltpu.make_async_copy(dst_hbm.at[pl.ds(0, _CH)], dbuf, sem_d).wait()

    def gather_issue(sbuf, rbuf, sem_g):
        pltpu.async_copy(y_hbm.at[sbuf], rbuf, sem_g)

    def gather_wait(sbuf, rbuf, sem_g):
        pltpu.make_async_copy(y_hbm.at[sbuf], rbuf, sem_g).wait()

    def count(dbuf):
        for g in range(_CH // 16):
            dv = dbuf[pl.ds(g * 16, 16)]
            for l in range(16):
                d = dv[l]
                hist[pl.ds(d, 16)] = hist[pl.ds(d, 16)] + one16

    pltpu.sync_copy(zrow_hbm, rows_a)
    for k in range(nz):
        pltpu.async_copy(rows_a, acc_sh.at[pl.ds(r0 + k * _CH, _CH)], sem_xa)
    for k in range(nz):
        pltpu.make_async_copy(rows_a, acc_sh.at[pl.ds(r0 + k * _CH, _CH)],
                              sem_xa).wait()

    pltpu.sync_copy(zflat_hbm, hist)
    plsc.subcore_barrier()

    idx_issue(0, sidx_a, didx_a, sem_sa, sem_da)
    idx_wait(sidx_a, didx_a, sem_sa, sem_da)
    gather_issue(sidx_a, rows_a, sem_ga)

    def pair(i, carry):
        a = 2 * i
        idx_issue(a + 1, sidx_b, didx_b, sem_sb, sem_db)
        gather_wait(sidx_a, rows_a, sem_ga)
        idx_wait(sidx_b, didx_b, sem_sb, sem_db)
        gather_issue(sidx_b, rows_b, sem_gb)
        pltpu.async_copy(rows_a, acc_sh.at[didx_a], sem_xa, add=True)
        count(didx_a)
        pltpu.make_async_copy(rows_a, acc_sh.at[didx_a], sem_xa).wait()
        idx_issue(jnp.minimum(a + 2, _NCH - 1), sidx_a, didx_a, sem_sa, sem_da)
        gather_wait(sidx_b, rows_b, sem_gb)
        idx_wait(sidx_a, didx_a, sem_sa, sem_da)
        gather_issue(sidx_a, rows_a, sem_ga)
        pltpu.async_copy(rows_b, acc_sh.at[didx_b], sem_xb, add=True)
        count(didx_b)
        pltpu.make_async_copy(rows_b, acc_sh.at[didx_b], sem_xb).wait()
        return carry

    lax.fori_loop(0, _NPAIR, pair, 0)
    gather_wait(sidx_a, rows_a, sem_ga)
    bt = ebase + _NCH * _CH
    pltpu.sync_copy(src_hbm.at[pl.ds(bt, _TL)], sidx_t)
    pltpu.sync_copy(dst_hbm.at[pl.ds(bt, _TL)], didx_t)
    pltpu.async_copy(y_hbm.at[sidx_t], rows_t, sem_ga).wait()
    pltpu.sync_copy(rows_t, acc_sh.at[didx_t], add=True)
    for l in range(_TL):
        dv = didx_t[pl.ds(0, 16)]
        d = dv[l]
        hist[pl.ds(d, 16)] = hist[pl.ds(d, 16)] + one16

    pltpu.sync_copy(hist.at[pl.ds(0, _NP)], stage_out.at[pl.ds(wid * _NP, _NP)])
    plsc.subcore_barrier()

    bufs = [rows_a, rows_b]
    sems = [sem_xa, sem_xb]
    for k in range(nz):
        bk = bufs[k % 2]
        if k >= 2:
            pltpu.make_async_copy(
                bk, acc_out.at[pl.ds(c * _NP + r0 + (k - 2) * _CH, _CH)],
                sems[k % 2]).wait()
        pltpu.sync_copy(acc_sh.at[pl.ds(r0 + k * _CH, _CH)], bk)
        pltpu.async_copy(bk, acc_out.at[pl.ds(c * _NP + r0 + k * _CH, _CH)],
                         sems[k % 2])
    for k in range(nz - 2, nz):
        pltpu.make_async_copy(
            bufs[k % 2], acc_out.at[pl.ds(c * _NP + r0 + k * _CH, _CH)],
            sems[k % 2]).wait()

    def rz(i, carry):
        racc[pl.ds(i * 16, 16)] = z16
        return carry

    lax.fori_loop(0, _RPT // 16, rz, 0)
    rbufs = [rtmp, rtmp2]
    rsems = [sem_ga, sem_gb]
    pltpu.async_copy(stage_out.at[pl.ds((0 * _NC + c) * _NP + r0, _RPT)],
                     rbufs[0], rsems[0])
    for t in range(_NS):
        rb = rbufs[t % 2]
        pltpu.make_async_copy(stage_out.at[pl.ds(0, _RPT)], rb,
                              rsems[t % 2]).wait()
        if t + 1 < _NS:
            twid = (t + 1) * _NC + c
            pltpu.async_copy(stage_out.at[pl.ds(twid * _NP + r0, _RPT)],
                             rbufs[(t + 1) % 2], rsems[(t + 1) % 2])

        def radd(i, carry, rb=rb):
            sl = pl.ds(i * 16, 16)
            racc[sl] = racc[sl] + rb[sl]
            return carry

        lax.fori_loop(0, _RPT // 16, radd, 0)
    pltpu.sync_copy(racc, cnt_out.at[pl.ds(c * _NP + r0, _RPT)])


def _dotT(x, w):
    return lax.dot_general(x, w, dimension_numbers=(((1,), (1,)), ((), ())),
                           preferred_element_type=jnp.float32)


def _linear_body(xa, xf, xi, xp, wa, wf, wi, wp, ba, bf, bi, bp,
                 w1, w2, w3, w4, w5, w6, w7,
                 oa, of, oi, op, y1, y2, y3, y4, y5, y6, y7):
    a, f, i, p = xa[...], xf[...], xi[...], xp[...]
    oa[...] = _dotT(a, wa[...]) + ba[...]
    of[...] = _dotT(f, wf[...]) + bf[...]
    oi[...] = _dotT(i, wi[...]) + bi[...]
    op[...] = _dotT(p, wp[...]) + bp[...]
    y1[...] = _dotT(a, w1[...])
    y2[...] = _dotT(i, w2[...])
    y3[...] = _dotT(a, w3[...])
    y4[...] = _dotT(p, w4[...])
    y5[...] = _dotT(p, w5[...])
    y6[...] = _dotT(p, w6[...])
    y7[...] = _dotT(f, w7[...])


_xspec = pl.BlockSpec((_BLK, _D), lambda i: (i, 0))
_wspec = pl.BlockSpec((_D, _D), lambda i: (0, 0))
_bspec = pl.BlockSpec((1, _D), lambda i: (0, 0))
_accspec = pl.BlockSpec((_NC, _BLK, _D), lambda i: (0, i, 0))
_oshape = jax.ShapeDtypeStruct((_N, _D), jnp.float32)

_linear_call = pl.pallas_call(
    _linear_body,
    grid=(_GRID,),
    in_specs=[_xspec] * 4 + [_wspec] * 4 + [_bspec] * 4 + [_wspec] * 7,
    out_specs=[_xspec] * 11,
    out_shape=[_oshape] * 11,
)


def _agg(acc_ref, cnt_ref):
    acc = acc_ref[...]
    total = acc[0] + acc[1]
    n = cnt_ref[0, :] + cnt_ref[1, :]
    return total / jnp.maximum(n, 1.0)[:, None]


def _epilogue_body(ra, rf, ri, rp, a1, a2, a3, a4, a5, a6, a7,
                   c1, c2, c3, c4, c5, c6, c7, oa, of, oi, op):
    oa[...] = ra[...] + _agg(a2, c2) + _agg(a4, c4)
    of[...] = rf[...] + _agg(a6, c6)
    oi[...] = ri[...] + _agg(a1, c1)
    op[...] = rp[...] + _agg(a3, c3) + _agg(a5, c5) + _agg(a7, c7)


_cntspec = pl.BlockSpec((_NC, _BLK), lambda i: (0, i))

_epilogue_call = pl.pallas_call(
    _epilogue_body,
    grid=(_GRID,),
    in_specs=[_xspec] * 4 + [_accspec] * 7 + [_cntspec] * 7,
    out_specs=[_xspec] * 4,
    out_shape=[_oshape] * 4,
)


def kernel(x_author, W_root_author, b_root_author,
           x_field_of_study, W_root_field_of_study, b_root_field_of_study,
           x_institution, W_root_institution, b_root_institution,
           x_paper, W_root_paper, b_root_paper,
           W_rel_author_affiliated_with_institution, ei_author_affiliated_with_institution,
           W_rel_institution_to_author, ei_institution_to_author,
           W_rel_author_writes_paper, ei_author_writes_paper,
           W_rel_paper_to_author, ei_paper_to_author,
           W_rel_paper_cites_paper, ei_paper_cites_paper,
           W_rel_paper_has_topic_field_of_study, ei_paper_has_topic_field_of_study,
           W_rel_field_of_study_to_paper, ei_field_of_study_to_paper):
    outs = _linear_call(
        x_author, x_field_of_study, x_institution, x_paper,
        W_root_author, W_root_field_of_study, W_root_institution, W_root_paper,
        b_root_author.reshape(1, _D), b_root_field_of_study.reshape(1, _D),
        b_root_institution.reshape(1, _D), b_root_paper.reshape(1, _D),
        W_rel_author_affiliated_with_institution, W_rel_institution_to_author,
        W_rel_author_writes_paper, W_rel_paper_to_author, W_rel_paper_cites_paper,
        W_rel_paper_has_topic_field_of_study, W_rel_field_of_study_to_paper,
    )
    roots = outs[:4]
    ys = outs[4:]
    eis = (ei_author_affiliated_with_institution, ei_institution_to_author,
           ei_author_writes_paper, ei_paper_to_author, ei_paper_cites_paper,
           ei_paper_has_topic_field_of_study, ei_field_of_study_to_paper)

    zrow = jnp.zeros((_CH, _D), jnp.float32)
    zflat = jnp.zeros((_HP,), jnp.float32)

    accs, cnts = [], []
    for y, ei in zip(ys, eis):
        acc, cnt, _ = _sc_segment_mean(y, ei[1], ei[0], zrow, zflat)
        accs.append(acc.reshape(_NC, _NP, _D))
        cnts.append(cnt.reshape(_NC, _NP))

    return tuple(_epilogue_call(*roots, *accs, *cnts))
